# baseline (device time: 149776 ns/iter reference)
import jax
import jax.numpy as jnp
from jax import lax
from jax.experimental import pallas as pl
from jax.experimental.pallas import tpu as pltpu

N_DEV = 8
DH = 64
R_HOPS = N_DEV // 2
L_HOPS = N_DEV - 1 - R_HOPS


def _ring2log(t):
    return jnp.where(t < 4, t, 11 - t)


def kernel(x, Wq, Wk, Wv, Wo):
    B, s, D = x.shape
    S = N_DEV * s
    H = Wq.shape[1] // DH
    BH = B * H

    Wq_h = Wq.reshape(D, H, DH).transpose(1, 0, 2).astype(jnp.bfloat16)
    Wk_h = Wk.reshape(D, H, DH).transpose(1, 0, 2).astype(jnp.bfloat16)
    Wv_h = Wv.reshape(D, H, DH).transpose(1, 0, 2).astype(jnp.bfloat16)
    Wo_h = Wo.reshape(H, DH, D).astype(jnp.bfloat16)

    inv = 1.0 / (10000.0 ** (jnp.arange(0, DH, 2, dtype=jnp.float32) / DH))
    pos = jnp.arange(S, dtype=jnp.float32)[:, None] * inv[None, :]
    cos = jnp.repeat(jnp.cos(pos), 2, axis=-1)
    sin = jnp.repeat(jnp.sin(pos), 2, axis=-1)
    row = jnp.arange(DH)[:, None]
    col = jnp.arange(DH)[None, :]
    P = ((col == row + 1) & (row % 2 == 0)).astype(jnp.float32) - (
        (col == row - 1) & (row % 2 == 1)
    ).astype(jnp.float32)

    def body(
        x_ref, wq_ref, wk_ref, wv_ref, wo_ref, cos_ref, sin_ref, p_ref,
        out_ref,
        xg, q_scr, k_scr, v_scr, partial,
        agbuf_r, agbuf_l, comm_r, comm_l,
        ag_sr, ag_rr, ag_sl, ag_rl, rs_sr, rs_rr, rs_sl, rs_rl,
    ):
        my = lax.axis_index("i")
        r = _ring2log(my)
        left = _ring2log(lax.rem(r + N_DEV - 1, N_DEV))
        right = _ring2log(lax.rem(r + 1, N_DEV))

        def c(off):
            return _ring2log(lax.rem(r + N_DEV + off, N_DEV))

        barrier = pltpu.get_barrier_semaphore()
        for nbr in (left, right):
            pl.semaphore_signal(
                barrier, inc=1, device_id=(nbr,),
                device_id_type=pl.DeviceIdType.MESH,
            )
        pl.semaphore_wait(barrier, 2)

        p_mat = p_ref[...]

        def proj_chunk(cc):
            rows = pl.ds(cc * s, s)
            cos_c = cos_ref[rows, :]
            sin_c = sin_ref[rows, :]

            def pbody(bh, _):
                b = bh // H
                h = lax.rem(bh, H)
                xb = xg[b, rows, :]
                q = jnp.dot(xb, wq_ref[h], preferred_element_type=jnp.float32)
                k = jnp.dot(xb, wk_ref[h], preferred_element_type=jnp.float32)
                v = jnp.dot(xb, wv_ref[h], preferred_element_type=jnp.float32)
                q = (q * cos_c + jnp.dot(q, p_mat) * sin_c) * 0.125
                k = k * cos_c + jnp.dot(k, p_mat) * sin_c
                q_scr[bh, rows, :] = q.astype(jnp.bfloat16)
                k_scr[bh, rows, :] = k.astype(jnp.bfloat16)
                v_scr[bh, rows, :] = v.astype(jnp.bfloat16)
                return 0

            lax.fori_loop(0, BH, pbody, 0)

        def attn_chunk(cc):
            rows = pl.ds(cc * s, s)

            for b in range(B):

                def abody(h, acc, b=b):
                    bh = b * H + h
                    q_blk = q_scr[bh, rows, :]
                    e = jnp.exp(
                        lax.dot_general(
                            q_blk, k_scr[bh],
                            (((1,), (1,)), ((), ())),
                            preferred_element_type=jnp.float32,
                        )
                    )
                    denom = jnp.sum(e, axis=1, keepdims=True)
                    ctx = jnp.dot(
                        e.astype(jnp.bfloat16), v_scr[bh],
                        preferred_element_type=jnp.float32,
                    )
                    ctx = (ctx / denom).astype(jnp.bfloat16)
                    return acc + jnp.dot(
                        ctx, wo_ref[h], preferred_element_type=jnp.float32
                    )

                partial[b, rows, :] = lax.fori_loop(
                    0, H, abody, jnp.zeros((s, D), jnp.float32)
                )

        def hop(buf, ssem, rsem, ss, rs_, target):
            return pltpu.make_async_remote_copy(
                src_ref=buf.at[ss],
                dst_ref=buf.at[rs_],
                send_sem=ssem.at[ss],
                recv_sem=rsem.at[rs_],
                device_id=(target,),
                device_id_type=pl.DeviceIdType.MESH,
            )

        xg[:, pl.ds(my * s, s), :] = x_ref[...]
        agbuf_r[0] = x_ref[...]
        agbuf_l[0] = x_ref[...]
        for h_ag in range(R_HOPS):
            ss, rs_ = h_ag % 2, (h_ag + 1) % 2
            rdma_r = hop(agbuf_r, ag_sr, ag_rr, ss, rs_, right)
            rdma_r.start()
            if h_ag < L_HOPS:
                rdma_l = hop(agbuf_l, ag_sl, ag_rl, ss, rs_, left)
                rdma_l.start()
            if h_ag == 0:
                proj_chunk(my)
            else:
                proj_chunk(c(-h_ag))
                proj_chunk(c(h_ag))
            rdma_r.wait()
            xg[:, pl.ds(c(-(h_ag + 1)) * s, s), :] = agbuf_r[rs_]
            if h_ag < L_HOPS:
                rdma_l.wait()
                xg[:, pl.ds(c(h_ag + 1) * s, s), :] = agbuf_l[rs_]
        proj_chunk(c(4))

        attn_chunk(c(4))
        attn_chunk(c(5))
        comm_r[0] = partial[:, pl.ds(c(4) * s, s), :].astype(jnp.bfloat16)
        comm_l[0] = partial[:, pl.ds(c(5) * s, s), :].astype(jnp.bfloat16)
        compute_sched = [(3, 6), (2, 7), (1, 0)]
        for st in range(R_HOPS):
            ss, rs_ = st % 2, (st + 1) % 2
            rdma_r = hop(comm_r, rs_sr, rs_rr, ss, rs_, right)
            rdma_r.start()
            if st < L_HOPS:
                rdma_l = hop(comm_l, rs_sl, rs_rl, ss, rs_, left)
                rdma_l.start()
            if st < len(compute_sched):
                attn_chunk(c(compute_sched[st][0]))
                attn_chunk(c(compute_sched[st][1]))
            rdma_r.wait()
            if st < R_HOPS - 1:
                cr = c(L_HOPS - st)
                comm_r[rs_] = (
                    comm_r[rs_].astype(jnp.float32)
                    + partial[:, pl.ds(cr * s, s), :]
                ).astype(jnp.bfloat16)
            if st < L_HOPS:
                rdma_l.wait()
                if st < L_HOPS - 1:
                    cl = c(st - 2)
                    comm_l[rs_] = (
                        comm_l[rs_].astype(jnp.float32)
                        + partial[:, pl.ds(cl * s, s), :]
                    ).astype(jnp.bfloat16)

        out_ref[...] = (
            partial[:, pl.ds(my * s, s), :]
            + comm_r[R_HOPS % 2].astype(jnp.float32)
            + comm_l[L_HOPS % 2].astype(jnp.float32)
        )

    bf = jnp.bfloat16
    return pl.pallas_call(
        body,
        out_shape=jax.ShapeDtypeStruct((B, s, D), jnp.float32),
        in_specs=[pl.BlockSpec(memory_space=pltpu.VMEM)] * 8,
        out_specs=pl.BlockSpec(memory_space=pltpu.VMEM),
        scratch_shapes=[
            pltpu.VMEM((B, S, D), bf),
            pltpu.VMEM((BH, S, DH), bf),
            pltpu.VMEM((BH, S, DH), bf),
            pltpu.VMEM((BH, S, DH), bf),
            pltpu.VMEM((B, S, D), jnp.float32),
            pltpu.VMEM((2, B, s, D), bf),
            pltpu.VMEM((2, B, s, D), bf),
            pltpu.VMEM((2, B, s, D), bf),
            pltpu.VMEM((2, B, s, D), bf),
            pltpu.SemaphoreType.DMA((2,)),
            pltpu.SemaphoreType.DMA((2,)),
            pltpu.SemaphoreType.DMA((2,)),
            pltpu.SemaphoreType.DMA((2,)),
            pltpu.SemaphoreType.DMA((2,)),
            pltpu.SemaphoreType.DMA((2,)),
            pltpu.SemaphoreType.DMA((2,)),
            pltpu.SemaphoreType.DMA((2,)),
        ],
        compiler_params=pltpu.CompilerParams(
            collective_id=0, vmem_limit_bytes=100 * 1024 * 1024
        ),
    )(
        x.astype(bf), Wq_h, Wk_h, Wv_h, Wo_h, cos, sin, P
    )


# device time: 143469 ns/iter; 1.0440x vs baseline; 1.0440x over previous
import jax
import jax.numpy as jnp
from jax import lax
from jax.experimental import pallas as pl
from jax.experimental.pallas import tpu as pltpu

N_DEV = 8
DH = 64
R_HOPS = N_DEV // 2
L_HOPS = N_DEV - 1 - R_HOPS


def _ring2log(t):
    return jnp.where(t < 4, t, 11 - t)


def kernel(x, Wq, Wk, Wv, Wo):
    B, s, D = x.shape
    S = N_DEV * s
    H = Wq.shape[1] // DH
    BH = B * H

    Wq_h = Wq.reshape(D, H, DH).transpose(1, 0, 2).astype(jnp.bfloat16)
    Wk_h = Wk.reshape(D, H, DH).transpose(1, 0, 2).astype(jnp.bfloat16)
    Wv_h = Wv.reshape(D, H, DH).transpose(1, 0, 2).astype(jnp.bfloat16)
    Wo_h = Wo.reshape(H, DH, D).astype(jnp.bfloat16)

    inv = 1.0 / (10000.0 ** (jnp.arange(0, DH, 2, dtype=jnp.float32) / DH))
    pos = jnp.arange(S, dtype=jnp.float32)[:, None] * inv[None, :]
    cos = jnp.repeat(jnp.cos(pos), 2, axis=-1)
    sin = jnp.repeat(jnp.sin(pos), 2, axis=-1)
    row = jnp.arange(DH)[:, None]
    col = jnp.arange(DH)[None, :]
    P = ((col == row + 1) & (row % 2 == 0)).astype(jnp.float32) - (
        (col == row - 1) & (row % 2 == 1)
    ).astype(jnp.float32)

    def body(
        x_ref, wq_ref, wk_ref, wv_ref, wo_ref, cos_ref, sin_ref, p_ref,
        out_ref,
        xg, q_scr, k_scr, v_scr, partial,
        agbuf_r, agbuf_l, comm_r, comm_l,
        ag_sr, ag_rr, ag_sl, ag_rl, rs_sr, rs_rr, rs_sl, rs_rl,
    ):
        my = lax.axis_index("i")
        r = _ring2log(my)
        left = _ring2log(lax.rem(r + N_DEV - 1, N_DEV))
        right = _ring2log(lax.rem(r + 1, N_DEV))

        def c(off):
            return _ring2log(lax.rem(r + N_DEV + off, N_DEV))

        barrier = pltpu.get_barrier_semaphore()
        for nbr in (left, right):
            pl.semaphore_signal(
                barrier, inc=1, device_id=(nbr,),
                device_id_type=pl.DeviceIdType.MESH,
            )
        pl.semaphore_wait(barrier, 2)

        p_mat = p_ref[...]

        def proj_chunk(cc):
            rows = pl.ds(cc * s, s)
            cos_c = cos_ref[rows, :]
            sin_c = sin_ref[rows, :]

            def pbody(bh, _):
                b = bh // H
                h = lax.rem(bh, H)
                xb = xg[b, rows, :]
                q = jnp.dot(xb, wq_ref[h], preferred_element_type=jnp.float32)
                k = jnp.dot(xb, wk_ref[h], preferred_element_type=jnp.float32)
                v = jnp.dot(xb, wv_ref[h], preferred_element_type=jnp.float32)
                q = (q * cos_c + jnp.dot(q, p_mat) * sin_c) * 0.125
                k = k * cos_c + jnp.dot(k, p_mat) * sin_c
                q_scr[bh, rows, :] = q.astype(jnp.bfloat16)
                k_scr[bh, rows, :] = k.astype(jnp.bfloat16)
                v_scr[bh, rows, :] = v.astype(jnp.bfloat16)
                return 0

            lax.fori_loop(0, BH, pbody, 0)

        def attn_chunk(cc):
            rows = pl.ds(cc * s, s)
            partial[:, rows, :] = jnp.zeros((B, s, D), jnp.float32)

            def abody(bh, _):
                b = bh // H
                h = lax.rem(bh, H)
                q_blk = q_scr[bh, rows, :]
                e = jnp.exp(
                    lax.dot_general(
                        q_blk, k_scr[bh],
                        (((1,), (1,)), ((), ())),
                        preferred_element_type=jnp.float32,
                    )
                )
                denom = jnp.sum(e, axis=1, keepdims=True)
                ctx = jnp.dot(
                    e.astype(jnp.bfloat16), v_scr[bh],
                    preferred_element_type=jnp.float32,
                )
                ctx = (ctx / denom).astype(jnp.bfloat16)
                contrib = jnp.dot(
                    ctx, wo_ref[h], preferred_element_type=jnp.float32
                )
                partial[b, rows, :] = partial[b, rows, :] + contrib
                return 0

            lax.fori_loop(0, BH, abody, 0)

        def hop(buf, ssem, rsem, ss, rs_, target):
            return pltpu.make_async_remote_copy(
                src_ref=buf.at[ss],
                dst_ref=buf.at[rs_],
                send_sem=ssem.at[ss],
                recv_sem=rsem.at[rs_],
                device_id=(target,),
                device_id_type=pl.DeviceIdType.MESH,
            )

        xg[:, pl.ds(my * s, s), :] = x_ref[...]
        agbuf_r[0] = x_ref[...]
        agbuf_l[0] = x_ref[...]
        for h_ag in range(R_HOPS):
            ss, rs_ = h_ag % 2, (h_ag + 1) % 2
            rdma_r = hop(agbuf_r, ag_sr, ag_rr, ss, rs_, right)
            rdma_r.start()
            if h_ag < L_HOPS:
                rdma_l = hop(agbuf_l, ag_sl, ag_rl, ss, rs_, left)
                rdma_l.start()
            if h_ag == 0:
                proj_chunk(my)
            else:
                proj_chunk(c(-h_ag))
                proj_chunk(c(h_ag))
            rdma_r.wait()
            xg[:, pl.ds(c(-(h_ag + 1)) * s, s), :] = agbuf_r[rs_]
            if h_ag < L_HOPS:
                rdma_l.wait()
                xg[:, pl.ds(c(h_ag + 1) * s, s), :] = agbuf_l[rs_]
        proj_chunk(c(4))

        attn_chunk(c(4))
        attn_chunk(c(5))
        comm_r[0] = partial[:, pl.ds(c(4) * s, s), :].astype(jnp.bfloat16)
        comm_l[0] = partial[:, pl.ds(c(5) * s, s), :].astype(jnp.bfloat16)
        compute_sched = [(3, 6), (2, 7), (1, 0)]
        for st in range(R_HOPS):
            ss, rs_ = st % 2, (st + 1) % 2
            rdma_r = hop(comm_r, rs_sr, rs_rr, ss, rs_, right)
            rdma_r.start()
            if st < L_HOPS:
                rdma_l = hop(comm_l, rs_sl, rs_rl, ss, rs_, left)
                rdma_l.start()
            if st < len(compute_sched):
                attn_chunk(c(compute_sched[st][0]))
                attn_chunk(c(compute_sched[st][1]))
            rdma_r.wait()
            if st < R_HOPS - 1:
                cr = c(L_HOPS - st)
                comm_r[rs_] = (
                    comm_r[rs_].astype(jnp.float32)
                    + partial[:, pl.ds(cr * s, s), :]
                ).astype(jnp.bfloat16)
            if st < L_HOPS:
                rdma_l.wait()
                if st < L_HOPS - 1:
                    cl = c(st - 2)
                    comm_l[rs_] = (
                        comm_l[rs_].astype(jnp.float32)
                        + partial[:, pl.ds(cl * s, s), :]
                    ).astype(jnp.bfloat16)

        out_ref[...] = (
            partial[:, pl.ds(my * s, s), :]
            + comm_r[R_HOPS % 2].astype(jnp.float32)
            + comm_l[L_HOPS % 2].astype(jnp.float32)
        )

    bf = jnp.bfloat16
    return pl.pallas_call(
        body,
        out_shape=jax.ShapeDtypeStruct((B, s, D), jnp.float32),
        in_specs=[pl.BlockSpec(memory_space=pltpu.VMEM)] * 8,
        out_specs=pl.BlockSpec(memory_space=pltpu.VMEM),
        scratch_shapes=[
            pltpu.VMEM((B, S, D), bf),
            pltpu.VMEM((BH, S, DH), bf),
            pltpu.VMEM((BH, S, DH), bf),
            pltpu.VMEM((BH, S, DH), bf),
            pltpu.VMEM((B, S, D), jnp.float32),
            pltpu.VMEM((2, B, s, D), bf),
            pltpu.VMEM((2, B, s, D), bf),
            pltpu.VMEM((2, B, s, D), bf),
            pltpu.VMEM((2, B, s, D), bf),
            pltpu.SemaphoreType.DMA((2,)),
            pltpu.SemaphoreType.DMA((2,)),
            pltpu.SemaphoreType.DMA((2,)),
            pltpu.SemaphoreType.DMA((2,)),
            pltpu.SemaphoreType.DMA((2,)),
            pltpu.SemaphoreType.DMA((2,)),
            pltpu.SemaphoreType.DMA((2,)),
            pltpu.SemaphoreType.DMA((2,)),
        ],
        compiler_params=pltpu.CompilerParams(
            collective_id=0, vmem_limit_bytes=100 * 1024 * 1024
        ),
    )(
        x.astype(bf), Wq_h, Wk_h, Wv_h, Wo_h, cos, sin, P
    )
